# Initial kernel scaffold; baseline (speedup 1.0000x reference)
import functools

import jax
import jax.numpy as jnp
from jax import lax
from jax.experimental import pallas as pl
from jax.experimental.pallas import tpu as pltpu
from jax.experimental.pallas import tpu_sc as plsc

D = 32
BATCH = 16384 * 50
NC = 2
NS = 16
NW = NC * NS
BPW = BATCH // NW
CH = 800
NCHUNK = BPW // CH

_mesh = plsc.VectorSubcoreMesh(core_axis_name="c", subcore_axis_name="s")


@functools.partial(
    pl.kernel,
    out_type=[
        jax.ShapeDtypeStruct((BATCH, D), jnp.float32),
        jax.ShapeDtypeStruct((NW, 16), jnp.float32),
    ],
    mesh=_mesh,
    scratch_types=[
        pltpu.VMEM((BPW,), jnp.int32),
        pltpu.VMEM((2, CH, D), jnp.float32),
        pltpu.VMEM((16,), jnp.float32),
        pltpu.SemaphoreType.DMA,
        pltpu.SemaphoreType.DMA,
        pltpu.SemaphoreType.DMA,
        pltpu.SemaphoreType.DMA,
    ],
)
def _embedding_gather(table_hbm, idx_hbm, out_hbm, psum_hbm,
                      idx_v, rows_v, acc_v, gsem0, gsem1, ssem0, ssem1):
    wid = lax.axis_index("s") * NC + lax.axis_index("c")
    base = wid * BPW

    pltpu.sync_copy(idx_hbm.at[pl.ds(base, BPW)], idx_v)

    gsems = (gsem0, gsem1)
    ssems = (ssem0, ssem1)

    def gather_start(g, b):
        return pltpu.async_copy(
            table_hbm.at[idx_v.at[pl.ds(g * CH, CH)]], rows_v.at[b], gsems[b])

    def accumulate(b, accs):
        @plsc.parallel_loop(0, CH, step=1, unroll=8, carry=accs)
        def new_accs(i, c):
            a0, a1 = c
            a0 = a0 + rows_v[b, i, pl.ds(0, 16)]
            a1 = a1 + rows_v[b, i, pl.ds(16, 16)]
            return (a0, a1)
        return new_accs

    zeros = jnp.zeros((16,), jnp.float32)
    accs = (zeros, zeros)
    store_h = [None, None]
    gh = gather_start(0, 0)
    for g in range(NCHUNK):
        b = g % 2
        gh.wait()
        if g + 1 < NCHUNK:
            if store_h[1 - b] is not None:
                store_h[1 - b].wait()
            gh = gather_start(g + 1, 1 - b)
        accs = accumulate(b, accs)
        store_h[b] = pltpu.async_copy(
            rows_v.at[b], out_hbm.at[pl.ds(base + g * CH, CH)], ssems[b])
    store_h[0].wait()
    store_h[1].wait()

    acc_v[...] = accs[0] + accs[1]
    pltpu.sync_copy(acc_v, psum_hbm.at[wid])


def kernel(table, x):
    idx = x.reshape(-1).astype(jnp.int32)
    out_flat, psum = _embedding_gather(table, idx)
    loss = jnp.sum(psum)
    return (loss, out_flat.reshape(x.shape + (D,)))


# speedup vs baseline: 1.1378x; 1.1378x over previous
import functools

import jax
import jax.numpy as jnp
from jax import lax
from jax.experimental import pallas as pl
from jax.experimental.pallas import tpu as pltpu
from jax.experimental.pallas import tpu_sc as plsc

D = 32
BATCH = 16384 * 50
NC = 2
NS = 16
NW = NC * NS
BPW = BATCH // NW
CH = 800
NCHUNK = BPW // CH

_mesh = plsc.VectorSubcoreMesh(core_axis_name="c", subcore_axis_name="s")


@functools.partial(
    pl.kernel,
    out_type=[
        jax.ShapeDtypeStruct((BATCH, D), jnp.float32),
        jax.ShapeDtypeStruct((NW, 16), jnp.float32),
    ],
    mesh=_mesh,
    compiler_params=pltpu.CompilerParams(use_tc_tiling_on_sc=False),
    scratch_types=[
        pltpu.VMEM((BPW,), jnp.int32),
        pltpu.VMEM((2, CH, D), jnp.float32),
        pltpu.VMEM((16,), jnp.float32),
        pltpu.SemaphoreType.DMA,
        pltpu.SemaphoreType.DMA,
        pltpu.SemaphoreType.DMA,
        pltpu.SemaphoreType.DMA,
    ],
)
def _embedding_gather(table_hbm, idx_hbm, out_hbm, psum_hbm,
                      idx_v, rows_v, acc_v, gsem0, gsem1, ssem0, ssem1):
    wid = lax.axis_index("s") * NC + lax.axis_index("c")
    base = wid * BPW

    pltpu.sync_copy(idx_hbm.at[pl.ds(base, BPW)], idx_v)

    gsems = (gsem0, gsem1)
    ssems = (ssem0, ssem1)

    def gather_start(g, b):
        return pltpu.async_copy(
            table_hbm.at[idx_v.at[pl.ds(g * CH, CH)]], rows_v.at[b], gsems[b])

    def accumulate(b, accs):
        @plsc.parallel_loop(0, CH, step=1, unroll=8, carry=accs)
        def new_accs(i, c):
            a0, a1 = c
            a0 = a0 + rows_v[b, i, pl.ds(0, 16)]
            a1 = a1 + rows_v[b, i, pl.ds(16, 16)]
            return (a0, a1)
        return new_accs

    zeros = jnp.zeros((16,), jnp.float32)
    accs = (zeros, zeros)
    store_h = [None, None]
    gh = gather_start(0, 0)
    for g in range(NCHUNK):
        b = g % 2
        gh.wait()
        if g + 1 < NCHUNK:
            if store_h[1 - b] is not None:
                store_h[1 - b].wait()
            gh = gather_start(g + 1, 1 - b)
        accs = accumulate(b, accs)
        store_h[b] = pltpu.async_copy(
            rows_v.at[b], out_hbm.at[pl.ds(base + g * CH, CH)], ssems[b])
    store_h[0].wait()
    store_h[1].wait()

    acc_v[...] = accs[0] + accs[1]
    pltpu.sync_copy(acc_v, psum_hbm.at[wid])


def kernel(table, x):
    idx = x.reshape(-1).astype(jnp.int32)
    out_flat, psum = _embedding_gather(table, idx)
    loss = jnp.sum(psum)
    return (loss, out_flat.reshape(x.shape + (D,)))


# speedup vs baseline: 1.9403x; 1.7053x over previous
import functools

import jax
import jax.numpy as jnp
from jax import lax
from jax.experimental import pallas as pl
from jax.experimental.pallas import tpu as pltpu
from jax.experimental.pallas import tpu_sc as plsc

D = 32
BATCH = 16384 * 50
NC = 2
NS = 16
NW = NC * NS
BPW = BATCH // NW
CH = 800
NCHUNK = BPW // CH

_mesh = plsc.VectorSubcoreMesh(core_axis_name="c", subcore_axis_name="s")


@functools.partial(
    pl.kernel,
    out_type=[
        jax.ShapeDtypeStruct((BATCH, D), jnp.float32),
        jax.ShapeDtypeStruct((NW, 16), jnp.float32),
    ],
    mesh=_mesh,
    compiler_params=pltpu.CompilerParams(use_tc_tiling_on_sc=False),
    scratch_types=[
        pltpu.VMEM((BPW,), jnp.int32),
        pltpu.VMEM((2, CH, D), jnp.float32),
        pltpu.VMEM((16,), jnp.float32),
        pltpu.SemaphoreType.DMA,
        pltpu.SemaphoreType.DMA,
        pltpu.SemaphoreType.DMA,
        pltpu.SemaphoreType.DMA,
    ],
)
def _embedding_gather(table_hbm, idx_hbm, out_hbm, psum_hbm,
                      idx_v, rows_v, acc_v, gsem0, gsem1, ssem0, ssem1):
    wid = lax.axis_index("s") * NC + lax.axis_index("c")
    base = wid * BPW

    pltpu.sync_copy(idx_hbm.at[pl.ds(base, BPW)], idx_v)

    gsems = (gsem0, gsem1)
    ssems = (ssem0, ssem1)

    def gather_start(g, b):
        return pltpu.async_copy(
            table_hbm.at[idx_v.at[pl.ds(g * CH, CH)]], rows_v.at[b], gsems[b])

    def accumulate(b, accs):
        @plsc.parallel_loop(0, CH, step=1, unroll=8, carry=accs)
        def new_accs(i, c):
            a0, a1 = c
            a0 = a0 + rows_v[b, i, pl.ds(0, 16)]
            a1 = a1 + rows_v[b, i, pl.ds(16, 16)]
            return (a0, a1)
        return new_accs

    zeros = jnp.zeros((16,), jnp.float32)
    accs = (zeros, zeros)
    store_h = [None, None]
    gh = gather_start(0, 0)
    for g in range(NCHUNK):
        b = g % 2
        gh.wait()
        if g + 1 < NCHUNK:
            if store_h[1 - b] is not None:
                store_h[1 - b].wait()
            gh = gather_start(g + 1, 1 - b)
        accs = accumulate(b, accs)
        store_h[b] = pltpu.async_copy(
            rows_v.at[b], out_hbm.at[pl.ds(base + g * CH, CH)], ssems[b])
    store_h[0].wait()
    store_h[1].wait()

    acc_v[...] = accs[0] + accs[1]
    pltpu.sync_copy(acc_v, psum_hbm.at[wid])


def kernel(table, x):
    n_b, n_j = x.shape
    idx = x.T.reshape(-1).astype(jnp.int32)
    out_flat, psum = _embedding_gather(table, idx)
    loss = jnp.sum(psum)
    out = out_flat.reshape(n_j, n_b, D).transpose(1, 0, 2)
    return (loss, out)
